# baseline (speedup 1.0000x reference)
import functools

import jax
import jax.numpy as jnp
from jax import lax
from jax.experimental import pallas as pl
from jax.experimental.pallas import tpu as pltpu
from jax.experimental.pallas import tpu_sc as plsc

_NUM_TYPES = 1000
_EMBED_DIM = 128
_HALF_DIM = _EMBED_DIM // 2
_BATCH = 16384

_NC = 2
_NS = 16
_NW = _NC * _NS
_B_PER_W = _BATCH // _NW
_CHUNK = 64
_NCHUNK = _B_PER_W // _CHUNK


_ROW_BLK = 200


def _fuse_table_body(embed_ref, chart_ref, cw_ref, cb_ref, w_ref,
                     comb_b_ref, out_ref):
    w1 = w_ref[:, :_EMBED_DIM]
    w2 = w_ref[:, _EMBED_DIM:]
    p = lax.dot_general(chart_ref[...], cw_ref[...], (((1,), (1,)), ((), ())),
                        preferred_element_type=jnp.float32)
    chart_part = lax.dot_general(p, w2, (((1,), (1,)), ((), ())),
                                 preferred_element_type=jnp.float32)
    base_part = lax.dot_general(embed_ref[...], w1,
                                (((1,), (1,)), ((), ())),
                                preferred_element_type=jnp.float32)
    bias = lax.dot_general(cb_ref[...], w2, (((1,), (1,)), ((), ())),
                           preferred_element_type=jnp.float32) + comb_b_ref[...]
    out_ref[...] = base_part + chart_part + bias


def _gather_body(table_hbm, idx_hbm, out_hbm, table_sh, idx_v, rows_v, gsem,
                 ssem):
    sid = lax.axis_index("s")
    wid = sid * _NC + lax.axis_index("c")
    base = wid * _B_PER_W

    @pl.when(sid == 0)
    def _stage():
        pltpu.sync_copy(table_hbm, table_sh)

    pltpu.sync_copy(idx_hbm.at[pl.ds(base, _B_PER_W)], idx_v)
    plsc.subcore_barrier()
    gathers = [
        pltpu.async_copy(table_sh.at[idx_v.at[pl.ds(j * _CHUNK, _CHUNK)]],
                         rows_v.at[pl.ds(j * _CHUNK, _CHUNK)], gsem)
        for j in range(_NCHUNK)
    ]
    scatters = []
    for j in range(_NCHUNK):
        gathers[j].wait()
        scatters.append(
            pltpu.async_copy(rows_v.at[pl.ds(j * _CHUNK, _CHUNK)],
                             out_hbm.at[pl.ds(base + j * _CHUNK, _CHUNK)],
                             ssem))
    for s in scatters:
        s.wait()


def kernel(type_id, embed_table, type_chart, chart_W, chart_b, combine_W,
           combine_b):
    fused = pl.pallas_call(
        _fuse_table_body,
        out_shape=jax.ShapeDtypeStruct((_NUM_TYPES, _EMBED_DIM), jnp.float32),
    )(embed_table, type_chart, chart_W, chart_b.reshape(1, _HALF_DIM),
      combine_W, combine_b.reshape(1, _EMBED_DIM))

    idx = type_id.astype(jnp.int32)

    mesh = plsc.VectorSubcoreMesh(core_axis_name="c", subcore_axis_name="s")
    gather = pl.kernel(
        _gather_body,
        out_type=jax.ShapeDtypeStruct((_BATCH, _EMBED_DIM), jnp.float32),
        mesh=mesh,
        scratch_types=[
            pltpu.MemorySpace.VMEM_SHARED((_NUM_TYPES, _EMBED_DIM),
                                          jnp.float32),
            pltpu.VMEM((_B_PER_W,), jnp.int32),
            pltpu.VMEM((_B_PER_W, _EMBED_DIM), jnp.float32),
            pltpu.SemaphoreType.DMA,
            pltpu.SemaphoreType.DMA,
        ],
    )
    return gather(fused, idx)


# speedup vs baseline: 1.0224x; 1.0224x over previous
import functools

import jax
import jax.numpy as jnp
from jax import lax
from jax.experimental import pallas as pl
from jax.experimental.pallas import tpu as pltpu
from jax.experimental.pallas import tpu_sc as plsc

_NUM_TYPES = 1000
_EMBED_DIM = 128
_HALF_DIM = _EMBED_DIM // 2
_BATCH = 16384

_NC = 2
_NS = 16
_NW = _NC * _NS
_B_PER_W = _BATCH // _NW
_CHUNK = 64
_NCHUNK = _B_PER_W // _CHUNK


_ROW_BLK = 200


_FBLK = 200
_NFBLK = _NUM_TYPES // _FBLK


def _fuse_table_body(embed_ref, chart_hbm, cw_ref, cb_ref, wt_ref,
                     comb_b_ref, out_ref, chart_buf, dsem):
    copies = [
        pltpu.make_async_copy(chart_hbm.at[pl.ds(i * _FBLK, _FBLK)],
                              chart_buf.at[i], dsem.at[i])
        for i in range(_NFBLK)
    ]
    for c in copies:
        c.start()
    w1t = wt_ref[:_EMBED_DIM, :]
    w2t = wt_ref[_EMBED_DIM:, :]
    bias = lax.dot_general(cb_ref[...], w2t, (((1,), (0,)), ((), ())),
                           preferred_element_type=jnp.float32) + comb_b_ref[...]
    base_part = lax.dot_general(embed_ref[...], w1t, (((1,), (0,)), ((), ())),
                                preferred_element_type=jnp.float32)
    for i in range(_NFBLK):
        copies[i].wait()
        p = lax.dot_general(chart_buf[i], cw_ref[...],
                            (((1,), (1,)), ((), ())),
                            preferred_element_type=jnp.float32)
        chart_part = lax.dot_general(p, w2t, (((1,), (0,)), ((), ())),
                                     preferred_element_type=jnp.float32)
        out_ref[pl.ds(i * _FBLK, _FBLK), :] = (
            base_part[i * _FBLK:(i + 1) * _FBLK, :] + chart_part + bias)


def _gather_body(table_hbm, idx_hbm, out_hbm, table_sh, idx_v, rows_v, gsem,
                 ssem):
    sid = lax.axis_index("s")
    wid = sid * _NC + lax.axis_index("c")
    base = wid * _B_PER_W

    @pl.when(sid == 0)
    def _stage():
        pltpu.sync_copy(table_hbm, table_sh)

    pltpu.sync_copy(idx_hbm.at[pl.ds(base, _B_PER_W)], idx_v)
    plsc.subcore_barrier()
    gathers = [
        pltpu.async_copy(table_sh.at[idx_v.at[pl.ds(j * _CHUNK, _CHUNK)]],
                         rows_v.at[pl.ds(j * _CHUNK, _CHUNK)], gsem)
        for j in range(_NCHUNK)
    ]
    scatters = []
    for j in range(_NCHUNK):
        gathers[j].wait()
        scatters.append(
            pltpu.async_copy(rows_v.at[pl.ds(j * _CHUNK, _CHUNK)],
                             out_hbm.at[pl.ds(base + j * _CHUNK, _CHUNK)],
                             ssem))
    for s in scatters:
        s.wait()


def kernel(type_id, embed_table, type_chart, chart_W, chart_b, combine_W,
           combine_b):
    fused = pl.pallas_call(
        _fuse_table_body,
        in_specs=[
            pl.BlockSpec(memory_space=pltpu.MemorySpace.VMEM),
            pl.BlockSpec(memory_space=pltpu.MemorySpace.HBM),
            pl.BlockSpec(memory_space=pltpu.MemorySpace.VMEM),
            pl.BlockSpec(memory_space=pltpu.MemorySpace.VMEM),
            pl.BlockSpec(memory_space=pltpu.MemorySpace.VMEM),
            pl.BlockSpec(memory_space=pltpu.MemorySpace.VMEM),
        ],
        scratch_shapes=[
            pltpu.VMEM((_NFBLK, _FBLK, _NUM_TYPES), jnp.float32),
            pltpu.SemaphoreType.DMA((_NFBLK,)),
        ],
        out_shape=jax.ShapeDtypeStruct((_NUM_TYPES, _EMBED_DIM), jnp.float32),
    )(embed_table, type_chart, chart_W, chart_b.reshape(1, _HALF_DIM),
      combine_W.T, combine_b.reshape(1, _EMBED_DIM))

    idx = type_id.astype(jnp.int32)

    mesh = plsc.VectorSubcoreMesh(core_axis_name="c", subcore_axis_name="s")
    gather = pl.kernel(
        _gather_body,
        out_type=jax.ShapeDtypeStruct((_BATCH, _EMBED_DIM), jnp.float32),
        mesh=mesh,
        scratch_types=[
            pltpu.MemorySpace.VMEM_SHARED((_NUM_TYPES, _EMBED_DIM),
                                          jnp.float32),
            pltpu.VMEM((_B_PER_W,), jnp.int32),
            pltpu.VMEM((_B_PER_W, _EMBED_DIM), jnp.float32),
            pltpu.SemaphoreType.DMA,
            pltpu.SemaphoreType.DMA,
        ],
    )
    return gather(fused, idx)


# speedup vs baseline: 1.0395x; 1.0167x over previous
import functools

import jax
import jax.numpy as jnp
from jax import lax
from jax.experimental import pallas as pl
from jax.experimental.pallas import tpu as pltpu
from jax.experimental.pallas import tpu_sc as plsc

_NUM_TYPES = 1000
_EMBED_DIM = 128
_HALF_DIM = _EMBED_DIM // 2
_BATCH = 16384

_NC = 2
_NS = 16
_NW = _NC * _NS
_B_PER_W = _BATCH // _NW
_CHUNK = 64
_NCHUNK = _B_PER_W // _CHUNK


_ROW_BLK = 200


def _fuse_table_body(embed_ref, chart_ref, cw_ref, cb_ref, wt_ref,
                     comb_b_ref, out_ref):
    w1t = wt_ref[:_EMBED_DIM, :]
    w2t = wt_ref[_EMBED_DIM:, :]
    p = lax.dot_general(chart_ref[...], cw_ref[...], (((1,), (1,)), ((), ())),
                        preferred_element_type=jnp.float32)
    chart_part = lax.dot_general(p, w2t, (((1,), (0,)), ((), ())),
                                 preferred_element_type=jnp.float32)
    base_part = lax.dot_general(embed_ref[...], w1t, (((1,), (0,)), ((), ())),
                                preferred_element_type=jnp.float32)
    bias = lax.dot_general(cb_ref[...], w2t, (((1,), (0,)), ((), ())),
                           preferred_element_type=jnp.float32) + comb_b_ref[...]
    out_ref[...] = base_part + chart_part + bias


def _gather_body(table_hbm, idx_hbm, out_hbm, table_sh, idx_v, rows_v, gsem,
                 ssem):
    sid = lax.axis_index("s")
    wid = sid * _NC + lax.axis_index("c")
    base = wid * _B_PER_W

    @pl.when(sid == 0)
    def _stage():
        pltpu.sync_copy(table_hbm, table_sh)

    pltpu.sync_copy(idx_hbm.at[pl.ds(base, _B_PER_W)], idx_v)
    plsc.subcore_barrier()

    def _chunk(j, carry):
        off = j * _CHUNK
        pltpu.async_copy(table_sh.at[idx_v.at[pl.ds(off, _CHUNK)]],
                         rows_v.at[pl.ds(off, _CHUNK)], gsem).wait()
        pltpu.async_copy(rows_v.at[pl.ds(off, _CHUNK)],
                         out_hbm.at[pl.ds(base + off, _CHUNK)], ssem)
        return carry

    lax.fori_loop(0, _NCHUNK, _chunk, 0)
    pltpu.make_async_copy(out_hbm.at[pl.ds(base, _B_PER_W)], rows_v,
                          ssem).wait()


def kernel(type_id, embed_table, type_chart, chart_W, chart_b, combine_W,
           combine_b):
    fused = pl.pallas_call(
        _fuse_table_body,
        out_shape=jax.ShapeDtypeStruct((_NUM_TYPES, _EMBED_DIM), jnp.float32),
    )(embed_table, type_chart, chart_W, chart_b.reshape(1, _HALF_DIM),
      combine_W.T, combine_b.reshape(1, _EMBED_DIM))

    idx = type_id.astype(jnp.int32)

    mesh = plsc.VectorSubcoreMesh(core_axis_name="c", subcore_axis_name="s")
    gather = pl.kernel(
        _gather_body,
        out_type=jax.ShapeDtypeStruct((_BATCH, _EMBED_DIM), jnp.float32),
        mesh=mesh,
        scratch_types=[
            pltpu.MemorySpace.VMEM_SHARED((_NUM_TYPES, _EMBED_DIM),
                                          jnp.float32),
            pltpu.VMEM((_B_PER_W,), jnp.int32),
            pltpu.VMEM((_B_PER_W, _EMBED_DIM), jnp.float32),
            pltpu.SemaphoreType.DMA,
            pltpu.SemaphoreType.DMA,
        ],
    )
    return gather(fused, idx)


# speedup vs baseline: 1.0604x; 1.0202x over previous
import functools

import jax
import jax.numpy as jnp
from jax import lax
from jax.experimental import pallas as pl
from jax.experimental.pallas import tpu as pltpu
from jax.experimental.pallas import tpu_sc as plsc

_NUM_TYPES = 1000
_EMBED_DIM = 128
_HALF_DIM = _EMBED_DIM // 2
_BATCH = 16384

_NC = 2
_NS = 16
_NW = _NC * _NS
_B_PER_W = _BATCH // _NW
_CHUNK = 64
_NCHUNK = _B_PER_W // _CHUNK


_ROW_BLK = 200


def _fuse_table_body(embed_ref, chart_ref, cw_ref, cb_ref, wt_ref,
                     comb_b_ref, out_ref):
    w1t = wt_ref[:_EMBED_DIM, :]
    w2t = wt_ref[_EMBED_DIM:, :]
    p = lax.dot_general(chart_ref[...], cw_ref[...], (((1,), (1,)), ((), ())),
                        preferred_element_type=jnp.float32)
    chart_part = lax.dot_general(p, w2t, (((1,), (0,)), ((), ())),
                                 preferred_element_type=jnp.float32)
    base_part = lax.dot_general(embed_ref[...], w1t, (((1,), (0,)), ((), ())),
                                preferred_element_type=jnp.float32)
    bias = lax.dot_general(cb_ref[...], w2t, (((1,), (0,)), ((), ())),
                           preferred_element_type=jnp.float32) + comb_b_ref[...]
    out_ref[...] = base_part + chart_part + bias


def _gather_body(table_hbm, idx_hbm, out_hbm, table_sh, idx_v, rows_v, gsem,
                 ssem):
    sid = lax.axis_index("s")
    wid = sid * _NC + lax.axis_index("c")
    base = wid * _B_PER_W

    @pl.when(sid == 0)
    def _stage():
        pltpu.sync_copy(table_hbm, table_sh)

    pltpu.sync_copy(idx_hbm.at[pl.ds(base, _B_PER_W)], idx_v)
    plsc.subcore_barrier()
    gathers = [
        pltpu.async_copy(table_sh.at[idx_v.at[pl.ds(j * _CHUNK, _CHUNK)]],
                         rows_v.at[pl.ds(j * _CHUNK, _CHUNK)], gsem)
        for j in range(_NCHUNK)
    ]
    scatters = []
    for j in range(_NCHUNK):
        gathers[j].wait()
        scatters.append(
            pltpu.async_copy(rows_v.at[pl.ds(j * _CHUNK, _CHUNK)],
                             out_hbm.at[pl.ds(base + j * _CHUNK, _CHUNK)],
                             ssem))
    for s in scatters:
        s.wait()


def kernel(type_id, embed_table, type_chart, chart_W, chart_b, combine_W,
           combine_b):
    fused = pl.pallas_call(
        _fuse_table_body,
        out_shape=jax.ShapeDtypeStruct((_NUM_TYPES, _EMBED_DIM), jnp.float32),
    )(embed_table, type_chart, chart_W, chart_b.reshape(1, _HALF_DIM),
      combine_W.T, combine_b.reshape(1, _EMBED_DIM))

    idx = type_id.astype(jnp.int32)

    mesh = plsc.VectorSubcoreMesh(core_axis_name="c", subcore_axis_name="s")
    gather = pl.kernel(
        _gather_body,
        out_type=jax.ShapeDtypeStruct((_BATCH, _EMBED_DIM), jnp.float32),
        mesh=mesh,
        scratch_types=[
            pltpu.MemorySpace.VMEM_SHARED((_NUM_TYPES, _EMBED_DIM),
                                          jnp.float32),
            pltpu.VMEM((_B_PER_W,), jnp.int32),
            pltpu.VMEM((_B_PER_W, _EMBED_DIM), jnp.float32),
            pltpu.SemaphoreType.DMA,
            pltpu.SemaphoreType.DMA,
        ],
    )
    return gather(fused, idx)
